# baseline (speedup 1.0000x reference)
import functools

import jax
import jax.numpy as jnp
from jax import lax
from jax.experimental import pallas as pl
from jax.experimental.pallas import tpu as pltpu
from jax.experimental.pallas import tpu_sc as plsc

N_NODES = 10000
D = 128
N_EDGES = 320000

NC, NS = 2, 16
CHUNK = 128
N_CHUNKS = N_EDGES // CHUNK
CHUNKS_SC0 = 79
SC0_CHUNKS = NS * CHUNKS_SC0
SC1_CHUNKS = N_CHUNKS - SC0_CHUNKS
ACC_ROWS = 10112

ROWS_PER_SUB = ACC_ROWS // NS


def _sc_aggregate(x, edges):
  mesh = plsc.VectorSubcoreMesh(core_axis_name="c", subcore_axis_name="s")

  @functools.partial(
      pl.kernel,
      out_type=jax.ShapeDtypeStruct((NC, ACC_ROWS, D), jnp.float32),
      mesh=mesh,
      scratch_types=[
          pltpu.VMEM((3, CHUNK), jnp.int32),
          pltpu.VMEM((3, CHUNK), jnp.int32),
          pltpu.VMEM((3, CHUNK, D), jnp.float32),
          pltpu.VMEM_SHARED((ACC_ROWS, D), jnp.float32),
          pltpu.SemaphoreType.DMA,
          pltpu.SemaphoreType.DMA,
          pltpu.SemaphoreType.DMA((3,)),
      ],
  )
  def agg(x_hbm, edges_hbm, out_hbm, s_idx, d_idx, rows, acc,
          sem_s, sem_d, sem_g):
    cid = lax.axis_index("c")
    sid = lax.axis_index("s")

    @pl.loop(0, CHUNK)
    def _(r):
      @pl.loop(0, D, step=16)
      def _(c):
        rows[0, r, pl.ds(c, 16)] = jnp.zeros((16,), jnp.float32)

    @pl.loop(0, ROWS_PER_SUB // CHUNK)
    def _(k):
      pltpu.sync_copy(
          rows.at[0],
          acc.at[pl.ds(sid * ROWS_PER_SUB + k * CHUNK, CHUNK)])

    pltpu.sync_copy(
        rows.at[0, pl.ds(0, ROWS_PER_SUB % CHUNK)],
        acc.at[pl.ds(sid * ROWS_PER_SUB + (ROWS_PER_SUB // CHUNK) * CHUNK,
                     ROWS_PER_SUB % CHUNK)])

    plsc.subcore_barrier()

    base_chunk = jnp.where(
        cid == 0, sid * CHUNKS_SC0,
        SC0_CHUNKS + sid * 77 + jnp.minimum(sid, 4))
    nchunks = jnp.where(cid == 0, CHUNKS_SC0,
                        jnp.where(sid < 4, 78, 77))
    base = base_chunk * CHUNK

    def idx_start(j, p):
      off = base + j * CHUNK
      pltpu.async_copy(edges_hbm.at[pl.ds(off, CHUNK)], s_idx.at[p], sem_s)
      pltpu.async_copy(edges_hbm.at[pl.ds(N_EDGES + off, CHUNK)],
                       d_idx.at[p], sem_d)

    def idx_wait(j, p):
      off = base + j * CHUNK
      pltpu.make_async_copy(edges_hbm.at[pl.ds(off, CHUNK)], s_idx.at[p],
                            sem_s).wait()
      pltpu.make_async_copy(edges_hbm.at[pl.ds(N_EDGES + off, CHUNK)],
                            d_idx.at[p], sem_d).wait()

    def gather_start(p):
      pltpu.async_copy(x_hbm.at[s_idx.at[p]], rows.at[p], sem_g.at[p])

    def gather_wait(p):
      pltpu.make_async_copy(x_hbm.at[s_idx.at[p]], rows.at[p],
                            sem_g.at[p]).wait()

    idx_start(0, 0)
    idx_wait(0, 0)
    gather_start(0)
    idx_start(1, 1)
    idx_wait(1, 1)
    gather_start(1)
    idx_start(2, 2)

    @pl.loop(0, nchunks)
    def _(j):
      p = lax.rem(j, 3)
      gather_wait(p)

      @pl.when(j + 2 < nchunks)
      def _():
        q = lax.rem(j + 2, 3)
        idx_wait(j + 2, q)
        gather_start(q)

      pltpu.sync_copy(rows.at[p], acc.at[d_idx.at[p]], add=True)

      @pl.when(j + 3 < nchunks)
      def _():
        idx_start(j + 3, p)

    plsc.subcore_barrier()

    rbase = sid * ROWS_PER_SUB
    pltpu.sync_copy(acc.at[pl.ds(rbase, ROWS_PER_SUB)],
                    out_hbm.at[cid, pl.ds(rbase, ROWS_PER_SUB)])

  return agg(x, edges)


def _tc_combine(partials, W):
  def body(p_ref, w_ref, o_ref):
    a = p_ref[0, :N_NODES] + p_ref[1, :N_NODES]
    o_ref[...] = lax.dot_general(
        a, w_ref[...], (((1,), (1,)), ((), ())),
        preferred_element_type=jnp.float32)

  return pl.pallas_call(
      body,
      out_shape=jax.ShapeDtypeStruct((N_NODES, D), jnp.float32),
  )(partials, W)


def kernel(x, W, edge_index, counts, out_edge_index, layer_i):
  del counts, out_edge_index, layer_i
  partials = _sc_aggregate(x, edge_index.reshape(-1))
  return _tc_combine(partials, W)
